# Initial kernel scaffold; baseline (speedup 1.0000x reference)
import jax
import jax.numpy as jnp
from jax.experimental import pallas as pl


def kernel(x, edge_index, batch, embed_table, Wl1, bl1, Wr1, Wl2, bl2, Wr2, Wlin, blin):
    raise NotImplementedError("write your pallas kernel here")


# speedup vs baseline: 4.4607x; 4.4607x over previous
import functools

import jax
import jax.numpy as jnp
from jax import lax
from jax.experimental import pallas as pl
from jax.experimental.pallas import tpu as pltpu
from jax.experimental.pallas import tpu_sc as plsc

N = 10000
E = 320000
VOCAB = 64
EMBED = 128
HIDDEN = 256
NCLS = 10
NGRAPH = 128

NC = 2
NS = 16
L = 16
CH = 128

NCHUNKS = E // CH
ROWS_PER_TILE = N // NS
ZR = 125


def _zero_vmem_2d(buf, rows, width):
  zeros = jnp.zeros((L,), jnp.float32)

  def body(r, carry):
    for j in range(width // L):
      buf[r, pl.ds(j * L, L)] = zeros
    return carry

  lax.fori_loop(0, rows, body, 0)


OHW = 128


def _hist_body(oh_hbm, src_hbm, dst_hbm, out_hbm,
               srcb, dstb, rowbuf, zbuf, acc, sem):
  c = lax.axis_index("c")
  s = lax.axis_index("s")

  _zero_vmem_2d(zbuf, ZR, OHW)
  for k in range(ROWS_PER_TILE // ZR):
    pltpu.sync_copy(zbuf, acc.at[pl.ds(s * ROWS_PER_TILE + k * ZR, ZR)])

  plsc.subcore_barrier()

  nch = jnp.where(s < 2, 79, 78)
  base = c * (NCHUNKS // NC) + s

  def chunk_body(i, carry):
    chunk = base + i * NS
    eb = chunk * CH
    pltpu.sync_copy(src_hbm.at[pl.ds(eb, CH)], srcb)
    pltpu.sync_copy(dst_hbm.at[pl.ds(eb, CH)], dstb)
    pltpu.async_copy(oh_hbm.at[srcb], rowbuf, sem).wait()
    pltpu.sync_copy(rowbuf, acc.at[dstb], add=True)
    return carry

  lax.fori_loop(0, nch, chunk_body, 0)

  plsc.subcore_barrier()

  row0 = s * ROWS_PER_TILE
  pltpu.sync_copy(acc.at[pl.ds(row0, ROWS_PER_TILE)],
                  out_hbm.at[c * NS + s])


def _sc_histogram(onehot, src, dst):
  mesh = plsc.VectorSubcoreMesh(core_axis_name="c", subcore_axis_name="s")
  k = pl.kernel(
      _hist_body,
      out_type=jax.ShapeDtypeStruct((NC * NS, ROWS_PER_TILE, OHW),
                                    jnp.float32),
      mesh=mesh,
      compiler_params=pltpu.CompilerParams(needs_layout_passes=False),
      scratch_types=[
          pltpu.VMEM((CH,), jnp.int32),
          pltpu.VMEM((CH,), jnp.int32),
          pltpu.VMEM((CH, OHW), jnp.float32),
          pltpu.VMEM((ZR, OHW), jnp.float32),
          pltpu.VMEM_SHARED((N, OHW), jnp.float32),
          pltpu.SemaphoreType.DMA,
      ],
  )
  return k(onehot, src, dst)


def _tc_oh_body(xi_ref, oh_ref):
  oh_ref[...] = (xi_ref[...] ==
                 lax.broadcasted_iota(jnp.int32, (ROWB_OH, OHW), 1)
                 ).astype(jnp.float32)


ROWB_OH = 2000


def _tc_onehot(xi2d):
  return pl.pallas_call(
      _tc_oh_body,
      grid=(N // ROWB_OH,),
      in_specs=[pl.BlockSpec((ROWB_OH, 1), lambda i: (i, 0))],
      out_specs=pl.BlockSpec((ROWB_OH, OHW), lambda i: (i, 0)),
      out_shape=jax.ShapeDtypeStruct((N, OHW), jnp.float32),
  )(xi2d)


def _segsum_body(m2_hbm, src_hbm, dst_hbm, out_hbm,
                 srcb, dstb, rowbuf, zbuf, acc, sem):
  c = lax.axis_index("c")
  s = lax.axis_index("s")

  _zero_vmem_2d(zbuf, ZR, EMBED)
  for k in range(ROWS_PER_TILE // ZR):
    pltpu.sync_copy(zbuf, acc.at[pl.ds(s * ROWS_PER_TILE + k * ZR, ZR)])

  plsc.subcore_barrier()

  nch = jnp.where(s < 4, 157, 156)
  coff = c * N

  def chunk_body(i, carry):
    chunk = s + i * NS
    eb = chunk * CH
    pltpu.sync_copy(src_hbm.at[pl.ds(eb, CH)], srcb)
    pltpu.sync_copy(dst_hbm.at[pl.ds(eb, CH)], dstb)
    for j in range(CH // L):
      srcb[pl.ds(j * L, L)] = srcb[pl.ds(j * L, L)] + coff
    pltpu.async_copy(m2_hbm.at[srcb], rowbuf, sem).wait()
    pltpu.sync_copy(rowbuf, acc.at[dstb], add=True)
    return carry

  lax.fori_loop(0, nch, chunk_body, 0)

  plsc.subcore_barrier()

  row0 = s * ROWS_PER_TILE
  pltpu.sync_copy(acc.at[pl.ds(row0, ROWS_PER_TILE)],
                  out_hbm.at[c * NS + s])


def _sc_segsum(m2flat, src, dst):
  mesh = plsc.VectorSubcoreMesh(core_axis_name="c", subcore_axis_name="s")
  k = pl.kernel(
      _segsum_body,
      out_type=jax.ShapeDtypeStruct((NC * NS, ROWS_PER_TILE, EMBED),
                                    jnp.float32),
      mesh=mesh,
      compiler_params=pltpu.CompilerParams(needs_layout_passes=False),
      scratch_types=[
          pltpu.VMEM((CH,), jnp.int32),
          pltpu.VMEM((CH,), jnp.int32),
          pltpu.VMEM((CH, EMBED), jnp.float32),
          pltpu.VMEM((ZR, EMBED), jnp.float32),
          pltpu.VMEM_SHARED((N, EMBED), jnp.float32),
          pltpu.SemaphoreType.DMA,
      ],
  )
  return k(m2flat, src, dst)


ROWB = 400
NROWB = N // ROWB


def _dotT(a, b):
  return lax.dot_general(a, b, (((1,), (1,)), ((), ())),
                         preferred_element_type=jnp.float32)


def _tc1_body(c2_ref, xi_ref, emb_ref, wl1_ref, bl1_ref, wr1_ref,
              wl2_ref, wr2_ref, bl2_ref, m2_ref, r2_ref, deg_ref):
  cb = (c2_ref[0] + c2_ref[1])[:, :VOCAB]
  deg = jnp.sum(cb, axis=1, keepdims=True)
  degc = jnp.maximum(deg, 1.0)
  cn = cb / degc

  b1 = _dotT(emb_ref[...], wl1_ref[...])
  b1r = _dotT(emb_ref[...], wr1_ref[...])

  onehot = (xi_ref[...] ==
            lax.broadcasted_iota(jnp.int32, (ROWB, VOCAB), 1)
            ).astype(jnp.float32)

  h1 = jnp.maximum(
      jnp.dot(cn, b1, preferred_element_type=jnp.float32)
      + jnp.dot(onehot, b1r, preferred_element_type=jnp.float32)
      + bl1_ref[...], 0.0)

  m2 = _dotT(h1, wl2_ref[...])
  r2 = _dotT(h1, wr2_ref[...]) + bl2_ref[...]

  m2_ref[0] = m2[:, :EMBED]
  m2_ref[1] = m2[:, EMBED:]
  r2_ref[...] = r2
  deg_ref[...] = deg


def _tc_layer1(c2, xi2d, emb, wl1, bl1, wr1, wl2, wr2, bl2):
  full = lambda shape: pl.BlockSpec(shape, lambda i: (0,) * len(shape))
  return pl.pallas_call(
      _tc1_body,
      grid=(NROWB,),
      in_specs=[
          pl.BlockSpec((NC, ROWB, OHW), lambda i: (0, i, 0)),
          pl.BlockSpec((ROWB, 1), lambda i: (i, 0)),
          full((VOCAB, EMBED)),
          full((HIDDEN, EMBED)),
          full((1, HIDDEN)),
          full((HIDDEN, EMBED)),
          full((HIDDEN, HIDDEN)),
          full((HIDDEN, HIDDEN)),
          full((1, HIDDEN)),
      ],
      out_specs=[
          pl.BlockSpec((NC, ROWB, EMBED), lambda i: (0, i, 0)),
          pl.BlockSpec((ROWB, HIDDEN), lambda i: (i, 0)),
          pl.BlockSpec((ROWB, 1), lambda i: (i, 0)),
      ],
      out_shape=[
          jax.ShapeDtypeStruct((NC, N, EMBED), jnp.float32),
          jax.ShapeDtypeStruct((N, HIDDEN), jnp.float32),
          jax.ShapeDtypeStruct((N, 1), jnp.float32),
      ],
  )(c2, xi2d, emb, wl1, bl1, wr1, wl2, wr2, bl2)


def _tc2_body(s2_ref, r2_ref, deg_ref, batch_ref, wlin_ref, blin_ref,
              out_ref, acc, cnt):
  i = pl.program_id(0)

  @pl.when(i == 0)
  def _init():
    acc[...] = jnp.zeros_like(acc)
    cnt[...] = jnp.zeros_like(cnt)

  s2 = jnp.concatenate([s2_ref[0], s2_ref[1]], axis=1)
  degc = jnp.maximum(deg_ref[...], 1.0)
  h2 = jnp.maximum(s2 / degc + r2_ref[...], 0.0)

  onehot_t = (jnp.reshape(batch_ref[...], (1, ROWB)) ==
              lax.broadcasted_iota(jnp.int32, (NGRAPH, ROWB), 0)
              ).astype(jnp.float32)

  acc[...] += jnp.dot(onehot_t, h2, preferred_element_type=jnp.float32)
  cnt[...] += jnp.sum(onehot_t, axis=1, keepdims=True)

  @pl.when(i == NROWB - 1)
  def _finish():
    pooled = acc[...] / jnp.maximum(cnt[...], 1.0)
    out_ref[...] = _dotT(pooled, wlin_ref[...]) + blin_ref[...]


def _tc_layer2(s2, r2, deg, batch2d, wlin, blin):
  full = lambda shape: pl.BlockSpec(shape, lambda i: (0,) * len(shape))
  return pl.pallas_call(
      _tc2_body,
      grid=(NROWB,),
      in_specs=[
          pl.BlockSpec((NC, ROWB, EMBED), lambda i: (0, i, 0)),
          pl.BlockSpec((ROWB, HIDDEN), lambda i: (i, 0)),
          pl.BlockSpec((ROWB, 1), lambda i: (i, 0)),
          pl.BlockSpec((ROWB, 1), lambda i: (i, 0)),
          full((NCLS, HIDDEN)),
          full((1, NCLS)),
      ],
      out_specs=pl.BlockSpec((NGRAPH, NCLS), lambda i: (0, 0)),
      out_shape=jax.ShapeDtypeStruct((NGRAPH, NCLS), jnp.float32),
      scratch_shapes=[
          pltpu.VMEM((NGRAPH, HIDDEN), jnp.float32),
          pltpu.VMEM((NGRAPH, 1), jnp.float32),
      ],
  )(s2, r2, deg, batch2d, wlin, blin)


def _hist_emu(src, dst, xi):
  half = E // NC
  outs = []
  for c in range(NC):
    oh = jax.nn.one_hot(xi[src[c * half:(c + 1) * half]], VOCAB,
                        dtype=jnp.float32)
    outs.append(jax.ops.segment_sum(oh, dst[c * half:(c + 1) * half],
                                    num_segments=N))
  return jnp.reshape(jnp.stack(outs), (NC * NS, ROWS_PER_TILE, VOCAB))


def _segsum_emu(m2flat, src, dst):
  outs = []
  for c in range(NC):
    msgs = m2flat[c * N + src]
    outs.append(jax.ops.segment_sum(msgs, dst, num_segments=N))
  return jnp.reshape(jnp.stack(outs), (NC * NS, ROWS_PER_TILE, EMBED))


@jax.jit
def kernel(x, edge_index, batch, embed_table, Wl1, bl1, Wr1, Wl2, bl2, Wr2,
           Wlin, blin):
  src = edge_index[0]
  dst = edge_index[1]
  xi = jnp.reshape(x, (N,))

  c2 = _sc_histogram(_tc_onehot(x), src, dst)

  m2, r2, deg = _tc_layer1(
      jnp.reshape(c2, (NC, N, OHW)),
      jnp.reshape(x, (N, 1)),
      embed_table, Wl1, jnp.reshape(bl1, (1, HIDDEN)), Wr1,
      Wl2, Wr2, jnp.reshape(bl2, (1, HIDDEN)))

  s2 = _sc_segsum(jnp.reshape(m2, (NC * N, EMBED)), src, dst)

  return _tc_layer2(
      jnp.reshape(s2, (NC, N, EMBED)), r2, deg,
      jnp.reshape(batch, (N, 1)), Wlin, jnp.reshape(blin, (1, NCLS)))


# speedup vs baseline: 5.9335x; 1.3302x over previous
import functools

import jax
import jax.numpy as jnp
from jax import lax
from jax.experimental import pallas as pl
from jax.experimental.pallas import tpu as pltpu
from jax.experimental.pallas import tpu_sc as plsc

N = 10000
E = 320000
VOCAB = 64
EMBED = 128
HIDDEN = 256
NCLS = 10
NGRAPH = 128

NC = 2
NS = 16
L = 16
CH = 128

NCHUNKS = E // CH
ROWS_PER_TILE = N // NS
ZR = 125
K = 2


def _zero_vmem_2d(buf, rows, width):
  zeros = jnp.zeros((L,), jnp.float32)

  def body(r, carry):
    for j in range(width // L):
      buf[r, pl.ds(j * L, L)] = zeros
    return carry

  lax.fori_loop(0, rows, body, 0)


def _zero_acc_slice(rowbuf, acc, row0):
  for b in range(K):
    _zero_vmem_2d(rowbuf.at[b], CH, rowbuf.shape[2])
  nfull = ROWS_PER_TILE // CH
  for j in range(nfull):
    pltpu.sync_copy(rowbuf.at[j % K], acc.at[pl.ds(row0 + j * CH, CH)])
  rem = ROWS_PER_TILE - nfull * CH
  pltpu.sync_copy(rowbuf.at[0, pl.ds(0, rem)],
                  acc.at[pl.ds(row0 + nfull * CH, rem)])


OHW = 128


def _gather_scatter_batch(table_hbm, src_hbm, dst_hbm, srcb, dstb, rowbuf,
                          acc, isem, gsem, ssem, chunk0, src_off, nk):
  ids = []
  for b in range(nk):
    eb = (chunk0 + b) * CH
    ids.append(pltpu.async_copy(src_hbm.at[pl.ds(src_off + eb, CH)],
                                srcb.at[b], isem))
    ids.append(pltpu.async_copy(dst_hbm.at[pl.ds(eb, CH)], dstb.at[b], isem))
  for d in ids:
    d.wait()
  gds = [pltpu.async_copy(table_hbm.at[srcb.at[b]], rowbuf.at[b], gsem)
         for b in range(nk)]
  for d in gds:
    d.wait()
  sds = [pltpu.async_copy(rowbuf.at[b], acc.at[dstb.at[b]], ssem, add=True)
         for b in range(nk)]
  for d in sds:
    d.wait()


def _hist_body(oh_hbm, src_hbm, dst_hbm, out_hbm,
               srcb, dstb, rowbuf, zbuf, acc, isem, gsem, ssem):
  c = lax.axis_index("c")
  s = lax.axis_index("s")

  _zero_vmem_2d(zbuf, ZR, OHW)
  for k in range(ROWS_PER_TILE // ZR):
    pltpu.sync_copy(zbuf, acc.at[pl.ds(s * ROWS_PER_TILE + k * ZR, ZR)])

  plsc.subcore_barrier()

  base = c * (NCHUNKS // NC) + s * 78 + jnp.minimum(s, 2)

  def batch_body(g, carry):
    _gather_scatter_batch(oh_hbm, src_hbm, dst_hbm, srcb, dstb, rowbuf,
                          acc, isem, gsem, ssem, base + g * K, 0, K)
    return carry

  lax.fori_loop(0, 78 // K, batch_body, 0)

  @pl.when(s < 2)
  def _tail():
    _gather_scatter_batch(oh_hbm, src_hbm, dst_hbm, srcb, dstb, rowbuf,
                          acc, isem, gsem, ssem, base + 78, 0, 1)

  plsc.subcore_barrier()

  row0 = s * ROWS_PER_TILE
  pltpu.sync_copy(acc.at[pl.ds(row0, ROWS_PER_TILE)],
                  out_hbm.at[c * NS + s])


def _sc_histogram(onehot, src2d, dst2d):
  mesh = plsc.VectorSubcoreMesh(core_axis_name="c", subcore_axis_name="s")
  k = pl.kernel(
      _hist_body,
      out_type=jax.ShapeDtypeStruct((NC * NS, ROWS_PER_TILE, OHW),
                                    jnp.float32),
      mesh=mesh,
      compiler_params=pltpu.CompilerParams(needs_layout_passes=False),
      scratch_types=[
          pltpu.VMEM((K, CH), jnp.int32),
          pltpu.VMEM((K, CH), jnp.int32),
          pltpu.VMEM((K, CH, OHW), jnp.float32),
          pltpu.VMEM((ZR, OHW), jnp.float32),
          pltpu.VMEM_SHARED((N, OHW), jnp.float32),
          pltpu.SemaphoreType.DMA,
          pltpu.SemaphoreType.DMA,
          pltpu.SemaphoreType.DMA,
      ],
  )
  return k(onehot, src2d, dst2d)


def _tc_oh_body(xi_ref, oh_ref):
  oh_ref[...] = (xi_ref[...] ==
                 lax.broadcasted_iota(jnp.int32, (ROWB_OH, OHW), 1)
                 ).astype(jnp.float32)


ROWB_OH = 2000


def _tc_onehot(xi2d):
  return pl.pallas_call(
      _tc_oh_body,
      grid=(N // ROWB_OH,),
      in_specs=[pl.BlockSpec((ROWB_OH, 1), lambda i: (i, 0))],
      out_specs=pl.BlockSpec((ROWB_OH, OHW), lambda i: (i, 0)),
      out_shape=jax.ShapeDtypeStruct((N, OHW), jnp.float32),
  )(xi2d)


def _segsum_body(m2_hbm, srcs_hbm, dst_hbm, out_hbm,
                 srcb, dstb, rowbuf, zbuf, acc, isem, gsem, ssem):
  c = lax.axis_index("c")
  s = lax.axis_index("s")

  _zero_vmem_2d(zbuf, ZR, EMBED)
  for k in range(ROWS_PER_TILE // ZR):
    pltpu.sync_copy(zbuf, acc.at[pl.ds(s * ROWS_PER_TILE + k * ZR, ZR)])

  plsc.subcore_barrier()

  base = s * 156 + jnp.minimum(s, 4)
  src_off = c * E

  def batch_body(g, carry):
    _gather_scatter_batch(m2_hbm, srcs_hbm, dst_hbm, srcb, dstb, rowbuf,
                          acc, isem, gsem, ssem, base + g * K, src_off, K)
    return carry

  lax.fori_loop(0, 156 // K, batch_body, 0)

  @pl.when(s < 4)
  def _tail():
    _gather_scatter_batch(m2_hbm, srcs_hbm, dst_hbm, srcb, dstb, rowbuf,
                          acc, isem, gsem, ssem, base + 156, src_off, 1)

  plsc.subcore_barrier()

  row0 = s * ROWS_PER_TILE
  pltpu.sync_copy(acc.at[pl.ds(row0, ROWS_PER_TILE)],
                  out_hbm.at[c * NS + s])


def _sc_segsum(m2flat, srcs_flat, dst):
  mesh = plsc.VectorSubcoreMesh(core_axis_name="c", subcore_axis_name="s")
  k = pl.kernel(
      _segsum_body,
      out_type=jax.ShapeDtypeStruct((NC * NS, ROWS_PER_TILE, EMBED),
                                    jnp.float32),
      mesh=mesh,
      compiler_params=pltpu.CompilerParams(needs_layout_passes=False),
      scratch_types=[
          pltpu.VMEM((K, CH), jnp.int32),
          pltpu.VMEM((K, CH), jnp.int32),
          pltpu.VMEM((K, CH, EMBED), jnp.float32),
          pltpu.VMEM((ZR, EMBED), jnp.float32),
          pltpu.VMEM_SHARED((N, EMBED), jnp.float32),
          pltpu.SemaphoreType.DMA,
          pltpu.SemaphoreType.DMA,
          pltpu.SemaphoreType.DMA,
      ],
  )
  return k(m2flat, srcs_flat, dst)


ROWB = 400
NROWB = N // ROWB


def _dotT(a, b):
  return lax.dot_general(a, b, (((1,), (1,)), ((), ())),
                         preferred_element_type=jnp.float32)


def _tc1_body(c2_ref, xi_ref, emb_ref, wl1_ref, bl1_ref, wr1_ref,
              wl2_ref, wr2_ref, bl2_ref, m2_ref, r2_ref, deg_ref):
  cb = (c2_ref[0] + c2_ref[1])[:, :VOCAB]
  deg = jnp.sum(cb, axis=1, keepdims=True)
  degc = jnp.maximum(deg, 1.0)
  cn = cb / degc

  b1 = _dotT(emb_ref[...], wl1_ref[...])
  b1r = _dotT(emb_ref[...], wr1_ref[...])

  onehot = (xi_ref[...] ==
            lax.broadcasted_iota(jnp.int32, (ROWB, VOCAB), 1)
            ).astype(jnp.float32)

  h1 = jnp.maximum(
      jnp.dot(cn, b1, preferred_element_type=jnp.float32)
      + jnp.dot(onehot, b1r, preferred_element_type=jnp.float32)
      + bl1_ref[...], 0.0)

  m2 = _dotT(h1, wl2_ref[...])
  r2 = _dotT(h1, wr2_ref[...]) + bl2_ref[...]

  m2_ref[0] = m2[:, :EMBED]
  m2_ref[1] = m2[:, EMBED:]
  r2_ref[...] = r2
  deg_ref[...] = deg


def _tc_layer1(c2, xi2d, emb, wl1, bl1, wr1, wl2, wr2, bl2):
  full = lambda shape: pl.BlockSpec(shape, lambda i: (0,) * len(shape))
  return pl.pallas_call(
      _tc1_body,
      grid=(NROWB,),
      in_specs=[
          pl.BlockSpec((NC, ROWB, OHW), lambda i: (0, i, 0)),
          pl.BlockSpec((ROWB, 1), lambda i: (i, 0)),
          full((VOCAB, EMBED)),
          full((HIDDEN, EMBED)),
          full((1, HIDDEN)),
          full((HIDDEN, EMBED)),
          full((HIDDEN, HIDDEN)),
          full((HIDDEN, HIDDEN)),
          full((1, HIDDEN)),
      ],
      out_specs=[
          pl.BlockSpec((NC, ROWB, EMBED), lambda i: (0, i, 0)),
          pl.BlockSpec((ROWB, HIDDEN), lambda i: (i, 0)),
          pl.BlockSpec((ROWB, 1), lambda i: (i, 0)),
      ],
      out_shape=[
          jax.ShapeDtypeStruct((NC, N, EMBED), jnp.float32),
          jax.ShapeDtypeStruct((N, HIDDEN), jnp.float32),
          jax.ShapeDtypeStruct((N, 1), jnp.float32),
      ],
  )(c2, xi2d, emb, wl1, bl1, wr1, wl2, wr2, bl2)


def _tc2_body(s2_ref, r2_ref, deg_ref, batch_ref, wlin_ref, blin_ref,
              out_ref, acc, cnt):
  i = pl.program_id(0)

  @pl.when(i == 0)
  def _init():
    acc[...] = jnp.zeros_like(acc)
    cnt[...] = jnp.zeros_like(cnt)

  s2 = jnp.concatenate([s2_ref[0], s2_ref[1]], axis=1)
  degc = jnp.maximum(deg_ref[...], 1.0)
  h2 = jnp.maximum(s2 / degc + r2_ref[...], 0.0)

  onehot_t = (jnp.reshape(batch_ref[...], (1, ROWB)) ==
              lax.broadcasted_iota(jnp.int32, (NGRAPH, ROWB), 0)
              ).astype(jnp.float32)

  acc[...] += jnp.dot(onehot_t, h2, preferred_element_type=jnp.float32)
  cnt[...] += jnp.sum(onehot_t, axis=1, keepdims=True)

  @pl.when(i == NROWB - 1)
  def _finish():
    pooled = acc[...] / jnp.maximum(cnt[...], 1.0)
    out_ref[...] = _dotT(pooled, wlin_ref[...]) + blin_ref[...]


def _tc_layer2(s2, r2, deg, batch2d, wlin, blin):
  full = lambda shape: pl.BlockSpec(shape, lambda i: (0,) * len(shape))
  return pl.pallas_call(
      _tc2_body,
      grid=(NROWB,),
      in_specs=[
          pl.BlockSpec((NC, ROWB, EMBED), lambda i: (0, i, 0)),
          pl.BlockSpec((ROWB, HIDDEN), lambda i: (i, 0)),
          pl.BlockSpec((ROWB, 1), lambda i: (i, 0)),
          pl.BlockSpec((ROWB, 1), lambda i: (i, 0)),
          full((NCLS, HIDDEN)),
          full((1, NCLS)),
      ],
      out_specs=pl.BlockSpec((NGRAPH, NCLS), lambda i: (0, 0)),
      out_shape=jax.ShapeDtypeStruct((NGRAPH, NCLS), jnp.float32),
      scratch_shapes=[
          pltpu.VMEM((NGRAPH, HIDDEN), jnp.float32),
          pltpu.VMEM((NGRAPH, 1), jnp.float32),
      ],
  )(s2, r2, deg, batch2d, wlin, blin)


@jax.jit
def kernel(x, edge_index, batch, embed_table, Wl1, bl1, Wr1, Wl2, bl2, Wr2,
           Wlin, blin):
  src = edge_index[0]
  dst = edge_index[1]
  srcs = jnp.concatenate([src, src + N])

  c2 = _sc_histogram(_tc_onehot(x), src, dst)

  m2, r2, deg = _tc_layer1(
      jnp.reshape(c2, (NC, N, OHW)),
      jnp.reshape(x, (N, 1)),
      embed_table, Wl1, jnp.reshape(bl1, (1, HIDDEN)), Wr1,
      Wl2, Wr2, jnp.reshape(bl2, (1, HIDDEN)))

  s2 = _sc_segsum(jnp.reshape(m2, (NC * N, EMBED)), srcs,
                  dst)

  return _tc_layer2(
      jnp.reshape(s2, (NC, N, EMBED)), r2, deg,
      jnp.reshape(batch, (N, 1)), Wlin, jnp.reshape(blin, (1, NCLS)))
